# baseline (speedup 1.0000x reference)
import jax
import jax.numpy as jnp
from jax import lax
from jax.experimental import pallas as pl
from jax.experimental.pallas import tpu as pltpu
from jax.experimental.pallas import tpu_sc as plsc

BATCH = 4096
SEQ = 200
HIDDEN = 64
NUM_WORKERS = 32
BT = 128
LANES = 16
NC = 2
ZPAD = BT + 1


def _sc_body(idx_ref, table_ref, posp_ref, out_ref,
             idx0, idx1, pos0, pos1, rows0, rows1, zb0, zb1,
             sem_g0, sem_g1, sem_o):
    w = lax.axis_index("s") * NC + lax.axis_index("c")

    idxv = (idx0, idx1)
    posv = (pos0, pos1)
    rows = (rows0, rows1)
    zb = (zb0, zb1)
    sem_g = (sem_g0, sem_g1)

    iota = lax.iota(jnp.int32, LANES)
    zerov = jnp.zeros((LANES,), jnp.int32)
    htv = [(g16 * LANES + iota) // 8 for g16 in range(4)]
    hiv = [(g16 * LANES + iota) % 8 for g16 in range(4)]

    def stage(s, buf):
        pltpu.sync_copy(idx_ref.at[w, pl.ds(s, 1), :], idxv[buf])
        pltpu.sync_copy(posp_ref.at[pl.ds(s, 1), :], posv[buf])
        pltpu.async_copy(table_ref.at[idxv[buf].at[0]],
                         rows[buf].at[0], sem_g[buf])

    def out_dma(s, buf):
        return pltpu.async_copy(
            zb[buf].at[:, :, :, pl.ds(0, BT)],
            out_ref.at[s, :, pl.ds(w, 1), :, :], sem_o)

    def out_wait(s, buf):
        pltpu.make_async_copy(
            zb[buf].at[:, :, :, pl.ds(0, BT)],
            out_ref.at[s, :, pl.ds(w, 1), :, :], sem_o).wait()

    def process(buf):
        @pl.loop(0, BT // LANES)
        def _tile(bg):
            b0 = bg * LANES
            for g16 in range(4):
                pvec = posv[buf][0, pl.ds(g16 * LANES, LANES)]
                vs = [rows[buf][0, b0 + i, pl.ds(g16 * LANES, LANES)]
                      for i in range(LANES)]
                ws = [v + pvec for v in vs]
                for i in range(LANES):
                    bsplat = jnp.full((LANES,), b0 + i, jnp.int32)
                    plsc.store_scatter(zb[buf],
                                       [htv[g16], zerov, hiv[g16], bsplat],
                                       ws[i])

    stage(0, 0)

    @pl.loop(0, SEQ // 2)
    def _outer(s2):
        for half in range(2):
            buf = half
            s = s2 * 2 + half

            @pl.when(s < SEQ - 1)
            def _prefetch():
                stage(s + 1, 1 - buf)

            pltpu.make_async_copy(table_ref.at[idxv[buf].at[0]],
                                  rows[buf].at[0], sem_g[buf]).wait()

            @pl.when(s >= 2)
            def _drain_out():
                out_wait(s - 2, buf)

            process(buf)
            out_dma(s, buf)

    out_wait(SEQ - 2, 0)
    out_wait(SEQ - 1, 1)


@jax.jit
def _embed(idx3, table, posp):
    mesh = plsc.VectorSubcoreMesh(core_axis_name="c", subcore_axis_name="s")
    f = pl.kernel(
        _sc_body,
        out_type=jax.ShapeDtypeStruct((SEQ, 8, NUM_WORKERS, 8, BT),
                                      jnp.float32),
        mesh=mesh,
        scratch_types=[
            pltpu.VMEM((1, BT), jnp.int32),
            pltpu.VMEM((1, BT), jnp.int32),
            pltpu.VMEM((1, HIDDEN), jnp.float32),
            pltpu.VMEM((1, HIDDEN), jnp.float32),
            pltpu.VMEM((1, BT, HIDDEN), jnp.float32),
            pltpu.VMEM((1, BT, HIDDEN), jnp.float32),
            pltpu.VMEM((8, 1, 8, ZPAD), jnp.float32),
            pltpu.VMEM((8, 1, 8, ZPAD), jnp.float32),
            pltpu.SemaphoreType.DMA,
            pltpu.SemaphoreType.DMA,
            pltpu.SemaphoreType.DMA,
        ],
        compiler_params=pltpu.CompilerParams(use_tc_tiling_on_sc=False,
                                             needs_layout_passes=False),
    )
    return f(idx3, table, posp)


def kernel(input_ids, word_embeddings, position_embeddings):
    idx3 = (input_ids.reshape(NUM_WORKERS, BT, SEQ)
            .transpose(0, 2, 1).astype(jnp.int32))
    posp = position_embeddings[:SEQ]
    z = _embed(idx3, word_embeddings, posp)
    return (z.transpose(2, 4, 0, 1, 3).reshape(BATCH, SEQ, HIDDEN))


# speedup vs baseline: 1.0448x; 1.0448x over previous
import jax
import jax.numpy as jnp
from jax import lax
from jax.experimental import pallas as pl
from jax.experimental.pallas import tpu as pltpu
from jax.experimental.pallas import tpu_sc as plsc

BATCH = 4096
SEQ = 200
HIDDEN = 64
NUM_WORKERS = 32
BT = 128
LANES = 16
NC = 2
ZPAD = BT + 1


def _sc_body(idx_ref, table_ref, posp_ref, out_ref,
             idx0, idx1, pos0, pos1, rows0, rows1, zb0, zb1,
             sem_g0, sem_g1, sem_o):
    w = lax.axis_index("s") * NC + lax.axis_index("c")

    idxv = (idx0, idx1)
    posv = (pos0, pos1)
    rows = (rows0, rows1)
    zb = (zb0, zb1)
    sem_g = (sem_g0, sem_g1)

    iota = lax.iota(jnp.int32, LANES)
    zerov = jnp.zeros((LANES,), jnp.int32)
    htv = [(g16 * LANES + iota) // 8 for g16 in range(4)]
    hiv = [(g16 * LANES + iota) % 8 for g16 in range(4)]

    def stage(s, buf):
        pltpu.sync_copy(idx_ref.at[w, pl.ds(s, 1), :], idxv[buf])
        pltpu.sync_copy(posp_ref.at[pl.ds(s, 1), :], posv[buf])
        pltpu.async_copy(table_ref.at[idxv[buf].at[0]],
                         rows[buf].at[0], sem_g[buf])

    def out_dma(s, buf):
        return pltpu.async_copy(
            zb[buf].at[:, :, :, pl.ds(0, BT)],
            out_ref.at[s, :, pl.ds(w, 1), :, :], sem_o)

    def out_wait(s, buf):
        pltpu.make_async_copy(
            zb[buf].at[:, :, :, pl.ds(0, BT)],
            out_ref.at[s, :, pl.ds(w, 1), :, :], sem_o).wait()

    def process(buf):
        @pl.loop(0, BT // LANES)
        def _tile(bg):
            b0 = bg * LANES
            for g16 in range(4):
                pvec = posv[buf][0, pl.ds(g16 * LANES, LANES)]
                vs = [rows[buf][0, b0 + i, pl.ds(g16 * LANES, LANES)]
                      for i in range(LANES)]
                ws = [v + pvec for v in vs]
                for i in range(LANES):
                    bsplat = jnp.full((LANES,), b0 + i, jnp.int32)
                    plsc.store_scatter(zb[buf],
                                       [htv[g16], zerov, hiv[g16], bsplat],
                                       ws[i])

    stage(0, 0)

    @pl.loop(0, SEQ // 2)
    def _outer(s2):
        for half in range(2):
            buf = half
            s = s2 * 2 + half

            @pl.when(s < SEQ - 1)
            def _prefetch():
                stage(s + 1, 1 - buf)

            pltpu.make_async_copy(table_ref.at[idxv[buf].at[0]],
                                  rows[buf].at[0], sem_g[buf]).wait()

            @pl.when(s >= 2)
            def _drain_out():
                out_wait(s - 2, buf)

            process(buf)
            out_dma(s, buf)

    out_wait(SEQ - 2, 0)
    out_wait(SEQ - 1, 1)


@jax.jit
def _embed(idx3, table, posp):
    mesh = plsc.VectorSubcoreMesh(core_axis_name="c", subcore_axis_name="s")
    f = pl.kernel(
        _sc_body,
        out_type=jax.ShapeDtypeStruct((SEQ, 8, NUM_WORKERS, 8, BT),
                                      jnp.float32),
        mesh=mesh,
        scratch_types=[
            pltpu.VMEM((1, BT), jnp.int32),
            pltpu.VMEM((1, BT), jnp.int32),
            pltpu.VMEM((1, BT), jnp.float32),
            pltpu.VMEM((1, BT), jnp.float32),
            pltpu.VMEM((1, BT, BT), jnp.float32),
            pltpu.VMEM((1, BT, BT), jnp.float32),
            pltpu.VMEM((8, 1, 8, ZPAD), jnp.float32),
            pltpu.VMEM((8, 1, 8, ZPAD), jnp.float32),
            pltpu.SemaphoreType.DMA,
            pltpu.SemaphoreType.DMA,
            pltpu.SemaphoreType.DMA,
        ],
        compiler_params=pltpu.CompilerParams(use_tc_tiling_on_sc=False,
                                             needs_layout_passes=False),
    )
    return f(idx3, table, posp)


def kernel(input_ids, word_embeddings, position_embeddings):
    idx3 = (input_ids.reshape(NUM_WORKERS, BT, SEQ)
            .transpose(0, 2, 1).astype(jnp.int32))
    posp = jnp.pad(position_embeddings[:SEQ],
                   ((0, 0), (0, BT - HIDDEN)))
    wpad = jnp.pad(word_embeddings, ((0, 0), (0, BT - HIDDEN)))
    z = _embed(idx3, wpad, posp)
    return (z.transpose(2, 4, 0, 1, 3).reshape(BATCH, SEQ, HIDDEN))
